# Initial kernel scaffold; baseline (speedup 1.0000x reference)
import jax
import jax.numpy as jnp
from jax.experimental import pallas as pl


def kernel(x):
    raise NotImplementedError("write your pallas kernel here")


# speedup vs baseline: 25.0916x; 25.0916x over previous
import functools

import jax
import jax.numpy as jnp
from jax.experimental import pallas as pl
from jax.experimental.pallas import tpu as pltpu

_KERNEL = 7
_PAD = 3
_TOPK_RATIO = 0.1

_G = 8
_NBIN = 16


def _shifted(v, d, axis, fill):
    n = v.shape[axis]
    if d == 0:
        return v
    pad = jnp.full_like(v, fill)
    if d > 0:
        a = jax.lax.slice_in_dim(v, d, n, axis=axis)
        b = jax.lax.slice_in_dim(pad, 0, d, axis=axis)
        return jax.lax.concatenate([a, b], axis)
    else:
        a = jax.lax.slice_in_dim(pad, 0, -d, axis=axis)
        b = jax.lax.slice_in_dim(v, 0, n + d, axis=axis)
        return jax.lax.concatenate([a, b], axis)


def _pool_scores(v):
    rmax = v
    rsum = v
    for d in range(1, _PAD + 1):
        rmax = jnp.maximum(rmax, _shifted(v, d, 2, -jnp.inf))
        rmax = jnp.maximum(rmax, _shifted(v, -d, 2, -jnp.inf))
        rsum = rsum + _shifted(v, d, 2, 0.0)
        rsum = rsum + _shifted(v, -d, 2, 0.0)
    cmax = rmax
    csum = rsum
    for d in range(1, _PAD + 1):
        cmax = jnp.maximum(cmax, _shifted(rmax, d, 1, -jnp.inf))
        cmax = jnp.maximum(cmax, _shifted(rmax, -d, 1, -jnp.inf))
        csum = csum + _shifted(rsum, d, 1, 0.0)
        csum = csum + _shifted(rsum, -d, 1, 0.0)
    return jnp.abs(cmax - csum * (1.0 / float(_KERNEL * _KERNEL)))


def _count_bins(ids):
    parts = []
    for b in range(_NBIN):
        parts.append(jnp.sum((ids == b).astype(jnp.int32)))
    return jnp.stack(parts).reshape(1, _NBIN)


def _score_body(x_ref, s_ref, h_ref):
    @pl.when(pl.program_id(0) == 0)
    def _():
        h_ref[...] = jnp.zeros_like(h_ref)

    s = _pool_scores(x_ref[...])
    s_ref[...] = s
    bits = jax.lax.bitcast_convert_type(s, jnp.int32)
    h_ref[...] += _count_bins(bits >> 27)


def _hist_body(lo_ref, s_ref, h_ref, *, shift):
    @pl.when(pl.program_id(0) == 0)
    def _():
        h_ref[...] = jnp.zeros_like(h_ref)

    bits = jax.lax.bitcast_convert_type(s_ref[...], jnp.int32)
    ids = (bits - lo_ref[0, 0]) >> shift
    h_ref[...] += _count_bins(ids)


def _mask_body(t_ref, x_ref, s_ref, o_ref):
    bits = jax.lax.bitcast_convert_type(s_ref[...], jnp.int32)
    o_ref[...] = jnp.where(bits >= t_ref[0, 0], x_ref[...], 0.0)


def _refine(hist, lo, k, shift):
    h = hist.reshape(_NBIN)
    c = jnp.cumsum(h[::-1])[::-1]
    bstar = jnp.sum((c >= k).astype(jnp.int32)) - 1
    kept_above = c[bstar] - h[bstar]
    return lo + (bstar << shift), k - kept_above


@jax.jit
def kernel(x):
    n, ch, hh, ww = x.shape
    imgs = n * ch
    xf = x.reshape(imgs, hh, ww)
    grid = (imgs // _G,)
    blk = pl.BlockSpec((_G, hh, ww), lambda i: (i, 0, 0))
    hspec = pl.BlockSpec((1, _NBIN), lambda i: (0, 0))
    sspec = pl.BlockSpec(memory_space=pltpu.SMEM)

    total = imgs * hh * ww
    k = jnp.int32(int(_TOPK_RATIO * total))

    s, h1 = pl.pallas_call(
        _score_body,
        grid=grid,
        in_specs=[blk],
        out_specs=[blk, hspec],
        out_shape=[
            jax.ShapeDtypeStruct((imgs, hh, ww), jnp.float32),
            jax.ShapeDtypeStruct((1, _NBIN), jnp.int32),
        ],
    )(xf)

    lo, k_rem = _refine(h1, jnp.int32(0), k, 27)

    for shift in (23, 19, 15, 11, 7, 3):
        h = pl.pallas_call(
            functools.partial(_hist_body, shift=shift),
            grid=grid,
            in_specs=[sspec, blk],
            out_specs=hspec,
            out_shape=jax.ShapeDtypeStruct((1, _NBIN), jnp.int32),
        )(lo.reshape(1, 1), s)
        lo, k_rem = _refine(h, lo, k_rem, shift)

    out = pl.pallas_call(
        _mask_body,
        grid=grid,
        in_specs=[sspec, blk, blk],
        out_specs=blk,
        out_shape=jax.ShapeDtypeStruct((imgs, hh, ww), jnp.float32),
    )(lo.reshape(1, 1), xf, s)

    return out.reshape(n, ch, hh, ww)


# speedup vs baseline: 29.0790x; 1.1589x over previous
import functools

import jax
import jax.numpy as jnp
from jax import lax
from jax.experimental import pallas as pl
from jax.experimental.pallas import tpu as pltpu
from jax.experimental.pallas import tpu_sc as plsc

_KERNEL = 7
_PAD = 3
_TOPK_RATIO = 0.1

_G = 8
_NBIN = 16


def _shifted(v, d, axis, fill):
    n = v.shape[axis]
    if d == 0:
        return v
    pad = jnp.full_like(v, fill)
    if d > 0:
        a = jax.lax.slice_in_dim(v, d, n, axis=axis)
        b = jax.lax.slice_in_dim(pad, 0, d, axis=axis)
        return jax.lax.concatenate([a, b], axis)
    else:
        a = jax.lax.slice_in_dim(pad, 0, -d, axis=axis)
        b = jax.lax.slice_in_dim(v, 0, n + d, axis=axis)
        return jax.lax.concatenate([a, b], axis)


def _pool_scores(v):
    rmax = v
    rsum = v
    for d in range(1, _PAD + 1):
        rmax = jnp.maximum(rmax, _shifted(v, d, 2, -jnp.inf))
        rmax = jnp.maximum(rmax, _shifted(v, -d, 2, -jnp.inf))
        rsum = rsum + _shifted(v, d, 2, 0.0)
        rsum = rsum + _shifted(v, -d, 2, 0.0)
    cmax = rmax
    csum = rsum
    for d in range(1, _PAD + 1):
        cmax = jnp.maximum(cmax, _shifted(rmax, d, 1, -jnp.inf))
        cmax = jnp.maximum(cmax, _shifted(rmax, -d, 1, -jnp.inf))
        csum = csum + _shifted(rsum, d, 1, 0.0)
        csum = csum + _shifted(rsum, -d, 1, 0.0)
    return jnp.abs(cmax - csum * (1.0 / float(_KERNEL * _KERNEL)))


def _count_bins(ids):
    parts = []
    for b in range(_NBIN):
        parts.append(jnp.sum((ids == b).astype(jnp.int32)))
    return jnp.stack(parts).reshape(1, _NBIN)


def _score_body(x_ref, s_ref, h_ref):
    @pl.when(pl.program_id(0) == 0)
    def _():
        h_ref[...] = jnp.zeros_like(h_ref)

    s = _pool_scores(x_ref[...])
    s_ref[...] = s
    bits = jax.lax.bitcast_convert_type(s, jnp.int32)
    h_ref[...] += _count_bins(bits >> 27)


def _hist_body(lo_ref, s_ref, h_ref, *, shift):
    @pl.when(pl.program_id(0) == 0)
    def _():
        h_ref[...] = jnp.zeros_like(h_ref)

    bits = jax.lax.bitcast_convert_type(s_ref[...], jnp.int32)
    ids = (bits - lo_ref[0, 0]) >> shift
    h_ref[...] += _count_bins(ids)


def _mask_body(t_ref, x_ref, s_ref, o_ref):
    bits = jax.lax.bitcast_convert_type(s_ref[...], jnp.int32)
    o_ref[...] = jnp.where(bits >= t_ref[0, 0], x_ref[...], 0.0)


def _refine(hist, lo, k, shift, nbin=_NBIN):
    h = hist.reshape(nbin)
    c = jnp.cumsum(h[::-1])[::-1]
    bstar = jnp.sum((c >= k).astype(jnp.int32)) - 1
    kept_above = c[bstar] - h[bstar]
    return lo + (bstar << shift), k - kept_above


_SCBIN = 4096
_LSTRIDE = _SCBIN + 2
_CB = 12544


def _make_sc_hist(total, shift):
    nw = 32
    chunk = total // nw
    nchunks = chunk // _CB
    assert chunk % _CB == 0 and chunk % 16 == 0
    mesh = plsc.VectorSubcoreMesh(core_axis_name="c", subcore_axis_name="s")

    @functools.partial(
        pl.kernel,
        mesh=mesh,
        compiler_params=pltpu.CompilerParams(needs_layout_passes=False),
        out_type=jax.ShapeDtypeStruct((nw * _SCBIN,), jnp.int32),
        scratch_types=[
            pltpu.VMEM((16 * _LSTRIDE,), jnp.int32),
            pltpu.VMEM((_CB,), jnp.float32),
            pltpu.VMEM((_CB,), jnp.float32),
            pltpu.VMEM((_SCBIN,), jnp.int32),
            pltpu.VMEM((16,), jnp.int32),
            pltpu.SemaphoreType.DMA,
            pltpu.SemaphoreType.DMA,
        ],
    )
    def sc_hist(s_hbm, lo_hbm, out_hbm, hist, bufa, bufb, merged, lov, sema, semb):
        wid = lax.axis_index("s") * 2 + lax.axis_index("c")
        base = wid * chunk

        zero16 = jnp.zeros((16,), jnp.int32)

        def zbody(i, _):
            hist[pl.ds(i * 16, 16)] = zero16
            return 0
        lax.fori_loop(0, (16 * _LSTRIDE) // 16, zbody, 0)

        pltpu.sync_copy(lo_hbm, lov)
        lo = lov[...]
        laneoff = lax.iota(jnp.int32, 16) * _LSTRIDE + 1
        ones = jnp.ones((16,), jnp.int32)

        def process(buf):
            def vbody(i, _):
                v = buf[pl.ds(i * 16, 16)]
                bits = lax.bitcast_convert_type(v, jnp.int32)
                ids = (bits - lo) >> shift
                idx = jnp.clip(ids, -1, _SCBIN) + laneoff
                plsc.addupdate_scatter(hist, [idx], ones)
                return 0
            lax.fori_loop(0, _CB // 16, vbody, 0)

        bufs = (bufa, bufb)
        sems = (sema, semb)
        pltpu.async_copy(s_hbm.at[pl.ds(base, _CB)], bufa, sema)

        for c in range(nchunks):
            cur = c % 2
            if c + 1 < nchunks:
                pltpu.async_copy(
                    s_hbm.at[pl.ds(base + (c + 1) * _CB, _CB)],
                    bufs[1 - cur], sems[1 - cur])
            pltpu.make_async_copy(
                s_hbm.at[pl.ds(base + c * _CB, _CB)], bufs[cur], sems[cur]
            ).wait()
            process(bufs[cur])

        def mbody(cgrp, _):
            acc = hist[pl.ds(1 + cgrp * 16, 16)]
            for l in range(1, 16):
                acc = acc + hist[pl.ds(l * _LSTRIDE + 1 + cgrp * 16, 16)]
            merged[pl.ds(cgrp * 16, 16)] = acc
            return 0
        lax.fori_loop(0, _SCBIN // 16, mbody, 0)

        pltpu.sync_copy(merged, out_hbm.at[pl.ds(wid * _SCBIN, _SCBIN)])

    return sc_hist


@jax.jit
def kernel(x):
    n, ch, hh, ww = x.shape
    imgs = n * ch
    xf = x.reshape(imgs, hh, ww)
    grid = (imgs // _G,)
    blk = pl.BlockSpec((_G, hh, ww), lambda i: (i, 0, 0))
    hspec = pl.BlockSpec((1, _NBIN), lambda i: (0, 0))
    sspec = pl.BlockSpec(memory_space=pltpu.SMEM)

    total = imgs * hh * ww
    k = jnp.int32(int(_TOPK_RATIO * total))

    s, h1 = pl.pallas_call(
        _score_body,
        grid=grid,
        in_specs=[blk],
        out_specs=[blk, hspec],
        out_shape=[
            jax.ShapeDtypeStruct((imgs, hh, ww), jnp.float32),
            jax.ShapeDtypeStruct((1, _NBIN), jnp.int32),
        ],
    )(xf)

    lo, k_rem = _refine(h1, jnp.int32(0), k, 27)

    s_flat = s.reshape(total)
    for shift in (15, 3):
        h = _make_sc_hist(total, shift)(s_flat, jnp.full((16,), lo, jnp.int32))
        lo, k_rem = _refine(h.reshape(32, _SCBIN).sum(0), lo, k_rem, shift,
                            nbin=_SCBIN)

    out = pl.pallas_call(
        _mask_body,
        grid=grid,
        in_specs=[sspec, blk, blk],
        out_specs=blk,
        out_shape=jax.ShapeDtypeStruct((imgs, hh, ww), jnp.float32),
    )(lo.reshape(1, 1), xf, s)

    return out.reshape(n, ch, hh, ww)


# speedup vs baseline: 32.6508x; 1.1228x over previous
import functools

import jax
import jax.numpy as jnp
from jax import lax
from jax.experimental import pallas as pl
from jax.experimental.pallas import tpu as pltpu
from jax.experimental.pallas import tpu_sc as plsc

_KERNEL = 7
_PAD = 3
_TOPK_RATIO = 0.1

_G = 8
_NBIN = 16


def _shifted(v, d, axis, fill):
    n = v.shape[axis]
    if d == 0:
        return v
    pad = jnp.full_like(v, fill)
    if d > 0:
        a = jax.lax.slice_in_dim(v, d, n, axis=axis)
        b = jax.lax.slice_in_dim(pad, 0, d, axis=axis)
        return jax.lax.concatenate([a, b], axis)
    else:
        a = jax.lax.slice_in_dim(pad, 0, -d, axis=axis)
        b = jax.lax.slice_in_dim(v, 0, n + d, axis=axis)
        return jax.lax.concatenate([a, b], axis)


def _pool_scores(v):
    rmax = v
    rsum = v
    for d in range(1, _PAD + 1):
        rmax = jnp.maximum(rmax, _shifted(v, d, 2, -jnp.inf))
        rmax = jnp.maximum(rmax, _shifted(v, -d, 2, -jnp.inf))
        rsum = rsum + _shifted(v, d, 2, 0.0)
        rsum = rsum + _shifted(v, -d, 2, 0.0)
    cmax = rmax
    csum = rsum
    for d in range(1, _PAD + 1):
        cmax = jnp.maximum(cmax, _shifted(rmax, d, 1, -jnp.inf))
        cmax = jnp.maximum(cmax, _shifted(rmax, -d, 1, -jnp.inf))
        csum = csum + _shifted(rsum, d, 1, 0.0)
        csum = csum + _shifted(rsum, -d, 1, 0.0)
    return jnp.abs(cmax - csum * (1.0 / float(_KERNEL * _KERNEL)))


def _count_bins(ids):
    parts = []
    for b in range(_NBIN):
        parts.append(jnp.sum((ids == b).astype(jnp.int32)))
    return jnp.stack(parts).reshape(1, _NBIN)


def _score_body(x_ref, s_ref, h_ref):
    @pl.when(pl.program_id(0) == 0)
    def _():
        h_ref[...] = jnp.zeros_like(h_ref)

    s = _pool_scores(x_ref[...])
    s_ref[...] = s
    bits = jax.lax.bitcast_convert_type(s, jnp.int32)
    h_ref[...] += _count_bins(bits >> 27)


def _hist_body(lo_ref, s_ref, h_ref, *, shift):
    @pl.when(pl.program_id(0) == 0)
    def _():
        h_ref[...] = jnp.zeros_like(h_ref)

    bits = jax.lax.bitcast_convert_type(s_ref[...], jnp.int32)
    ids = (bits - lo_ref[0, 0]) >> shift
    h_ref[...] += _count_bins(ids)


def _mask_body(t_ref, x_ref, s_ref, o_ref):
    bits = jax.lax.bitcast_convert_type(s_ref[...], jnp.int32)
    o_ref[...] = jnp.where(bits >= t_ref[0, 0], x_ref[...], 0.0)


def _refine(hist, lo, k, shift, nbin=_NBIN):
    h = hist.reshape(nbin)
    c = jnp.cumsum(h[::-1])[::-1]
    bstar = jnp.sum((c >= k).astype(jnp.int32)) - 1
    kept_above = c[bstar] - h[bstar]
    return lo + (bstar << shift), k - kept_above


_SCBIN = 4096
_LSTRIDE = _SCBIN + 2
_CB = 18816
_UNROLL = 8


def _make_sc_hist(total, shift):
    nw = 32
    chunk = total // nw
    nchunks = chunk // _CB
    assert chunk % _CB == 0 and chunk % 16 == 0
    mesh = plsc.VectorSubcoreMesh(core_axis_name="c", subcore_axis_name="s")

    @functools.partial(
        pl.kernel,
        mesh=mesh,
        compiler_params=pltpu.CompilerParams(needs_layout_passes=False),
        out_type=jax.ShapeDtypeStruct((nw * _SCBIN,), jnp.int32),
        scratch_types=[
            pltpu.VMEM((16 * _LSTRIDE,), jnp.int32),
            pltpu.VMEM((_CB,), jnp.float32),
            pltpu.VMEM((_CB,), jnp.float32),
            pltpu.VMEM((_SCBIN,), jnp.int32),
            pltpu.VMEM((16,), jnp.int32),
            pltpu.SemaphoreType.DMA,
            pltpu.SemaphoreType.DMA,
        ],
    )
    def sc_hist(s_hbm, lo_hbm, out_hbm, hist, bufa, bufb, merged, lov, sema, semb):
        wid = lax.axis_index("s") * 2 + lax.axis_index("c")
        base = wid * chunk

        zero16 = jnp.zeros((16,), jnp.int32)

        def zbody(i, _):
            for u in range(6):
                hist[pl.ds((i * 6 + u) * 16, 16)] = zero16
            return 0
        lax.fori_loop(0, (16 * _LSTRIDE) // (16 * 6), zbody, 0)

        pltpu.sync_copy(lo_hbm, lov)
        lo = lov[...]
        laneoff = lax.iota(jnp.int32, 16) * _LSTRIDE
        ones = jnp.ones((16,), jnp.int32)
        ovf = jnp.full((16,), _SCBIN + 1, jnp.uint32)

        def process(buf):
            def vbody(i, _):
                for u in range(_UNROLL):
                    v = buf[pl.ds((i * _UNROLL + u) * 16, 16)]
                    bits = lax.bitcast_convert_type(v, jnp.int32)
                    rel = lax.bitcast_convert_type(bits - lo, jnp.uint32)
                    idu = jnp.minimum(rel >> shift, ovf)
                    idx = lax.bitcast_convert_type(idu, jnp.int32) + laneoff
                    plsc.addupdate_scatter(hist, [idx], ones)
                return 0
            lax.fori_loop(0, _CB // (16 * _UNROLL), vbody, 0)

        bufs = (bufa, bufb)
        sems = (sema, semb)
        pltpu.async_copy(s_hbm.at[pl.ds(base, _CB)], bufa, sema)

        for c in range(nchunks):
            cur = c % 2
            if c + 1 < nchunks:
                pltpu.async_copy(
                    s_hbm.at[pl.ds(base + (c + 1) * _CB, _CB)],
                    bufs[1 - cur], sems[1 - cur])
            pltpu.make_async_copy(
                s_hbm.at[pl.ds(base + c * _CB, _CB)], bufs[cur], sems[cur]
            ).wait()
            process(bufs[cur])

        def mbody(cgrp, _):
            acc = hist[pl.ds(cgrp * 16, 16)]
            for l in range(1, 16):
                acc = acc + hist[pl.ds(l * _LSTRIDE + cgrp * 16, 16)]
            merged[pl.ds(cgrp * 16, 16)] = acc
            return 0
        lax.fori_loop(0, _SCBIN // 16, mbody, 0)

        pltpu.sync_copy(merged, out_hbm.at[pl.ds(wid * _SCBIN, _SCBIN)])

    return sc_hist


@jax.jit
def kernel(x):
    n, ch, hh, ww = x.shape
    imgs = n * ch
    xf = x.reshape(imgs, hh, ww)
    grid = (imgs // _G,)
    blk = pl.BlockSpec((_G, hh, ww), lambda i: (i, 0, 0))
    hspec = pl.BlockSpec((1, _NBIN), lambda i: (0, 0))
    sspec = pl.BlockSpec(memory_space=pltpu.SMEM)

    total = imgs * hh * ww
    k = jnp.int32(int(_TOPK_RATIO * total))

    s, h1 = pl.pallas_call(
        _score_body,
        grid=grid,
        in_specs=[blk],
        out_specs=[blk, hspec],
        out_shape=[
            jax.ShapeDtypeStruct((imgs, hh, ww), jnp.float32),
            jax.ShapeDtypeStruct((1, _NBIN), jnp.int32),
        ],
    )(xf)

    lo, k_rem = _refine(h1, jnp.int32(0), k, 27)

    s_flat = s.reshape(total)
    for shift in (15, 3):
        h = _make_sc_hist(total, shift)(s_flat, jnp.full((16,), lo, jnp.int32))
        lo, k_rem = _refine(h.reshape(32, _SCBIN).sum(0), lo, k_rem, shift,
                            nbin=_SCBIN)

    out = pl.pallas_call(
        _mask_body,
        grid=grid,
        in_specs=[sspec, blk, blk],
        out_specs=blk,
        out_shape=jax.ShapeDtypeStruct((imgs, hh, ww), jnp.float32),
    )(lo.reshape(1, 1), xf, s)

    return out.reshape(n, ch, hh, ww)


# speedup vs baseline: 53.7950x; 1.6476x over previous
import functools

import jax
import jax.numpy as jnp
from jax import lax
from jax.experimental import pallas as pl
from jax.experimental.pallas import tpu as pltpu
from jax.experimental.pallas import tpu_sc as plsc

_KERNEL = 7
_PAD = 3
_TOPK_RATIO = 0.1

_G = 8
_NBIN = 16


def _shifted(v, d, axis, fill):
    n = v.shape[axis]
    if d == 0:
        return v
    pad = jnp.full_like(v, fill)
    if d > 0:
        a = jax.lax.slice_in_dim(v, d, n, axis=axis)
        b = jax.lax.slice_in_dim(pad, 0, d, axis=axis)
        return jax.lax.concatenate([a, b], axis)
    else:
        a = jax.lax.slice_in_dim(pad, 0, -d, axis=axis)
        b = jax.lax.slice_in_dim(v, 0, n + d, axis=axis)
        return jax.lax.concatenate([a, b], axis)


def _pool_scores(v):
    rmax = v
    rsum = v
    for d in range(1, _PAD + 1):
        rmax = jnp.maximum(rmax, _shifted(v, d, 2, -jnp.inf))
        rmax = jnp.maximum(rmax, _shifted(v, -d, 2, -jnp.inf))
        rsum = rsum + _shifted(v, d, 2, 0.0)
        rsum = rsum + _shifted(v, -d, 2, 0.0)
    cmax = rmax
    csum = rsum
    for d in range(1, _PAD + 1):
        cmax = jnp.maximum(cmax, _shifted(rmax, d, 1, -jnp.inf))
        cmax = jnp.maximum(cmax, _shifted(rmax, -d, 1, -jnp.inf))
        csum = csum + _shifted(rsum, d, 1, 0.0)
        csum = csum + _shifted(rsum, -d, 1, 0.0)
    return jnp.abs(cmax - csum * (1.0 / float(_KERNEL * _KERNEL)))


def _count_bins(ids):
    parts = []
    for b in range(_NBIN):
        parts.append(jnp.sum((ids == b).astype(jnp.int32)))
    return jnp.stack(parts).reshape(1, _NBIN)


def _score_body(x_ref, s_ref, h_ref):
    @pl.when(pl.program_id(0) == 0)
    def _():
        h_ref[...] = jnp.zeros_like(h_ref)

    s = _pool_scores(x_ref[...])
    s_ref[...] = s
    bits = jax.lax.bitcast_convert_type(s, jnp.int32)
    h_ref[...] += _count_bins(bits >> 27)


def _hist_body(lo_ref, s_ref, h_ref, *, shift):
    @pl.when(pl.program_id(0) == 0)
    def _():
        h_ref[...] = jnp.zeros_like(h_ref)

    bits = jax.lax.bitcast_convert_type(s_ref[...], jnp.int32)
    ids = (bits - lo_ref[0, 0]) >> shift
    h_ref[...] += _count_bins(ids)


def _mask_body(t_ref, x_ref, s_ref, o_ref):
    bits = jax.lax.bitcast_convert_type(s_ref[...], jnp.int32)
    o_ref[...] = jnp.where(bits >= t_ref[0, 0], x_ref[...], 0.0)


def _refine(hist, lo, k, shift, nbin=_NBIN):
    h = hist.reshape(nbin)
    c = jnp.cumsum(h[::-1])[::-1]
    bstar = jnp.sum((c >= k).astype(jnp.int32)) - 1
    kept_above = c[bstar] - h[bstar]
    return lo + (bstar << shift), k - kept_above


_SCBIN = 4096
_LSTRIDE = _SCBIN + 2
_CB = 18816
_UNROLL = 8


def _make_sc_hist(total, shift):
    nw = 32
    chunk = total // nw
    nchunks = chunk // _CB
    assert chunk % _CB == 0 and chunk % 16 == 0
    mesh = plsc.VectorSubcoreMesh(core_axis_name="c", subcore_axis_name="s")

    @functools.partial(
        pl.kernel,
        mesh=mesh,
        compiler_params=pltpu.CompilerParams(needs_layout_passes=False),
        out_type=jax.ShapeDtypeStruct((nw * _SCBIN,), jnp.int32),
        scratch_types=[
            pltpu.VMEM((16 * _LSTRIDE,), jnp.int32),
            pltpu.VMEM((_CB,), jnp.float32),
            pltpu.VMEM((_CB,), jnp.float32),
            pltpu.VMEM((_SCBIN,), jnp.int32),
            pltpu.VMEM((16,), jnp.int32),
            pltpu.SemaphoreType.DMA,
            pltpu.SemaphoreType.DMA,
        ],
    )
    def sc_hist(s_hbm, lo_hbm, out_hbm, hist, bufa, bufb, merged, lov, sema, semb):
        wid = lax.axis_index("s") * 2 + lax.axis_index("c")
        base = wid * chunk

        zero16 = jnp.zeros((16,), jnp.int32)

        @plsc.parallel_loop(0, (16 * _LSTRIDE) // 16, unroll=8)
        def _(i):
            hist[pl.ds(i * 16, 16)] = zero16

        pltpu.sync_copy(lo_hbm, lov)
        lo = lov[...]
        laneoff = lax.iota(jnp.int32, 16) * _LSTRIDE
        ones = jnp.ones((16,), jnp.int32)
        ovf = jnp.full((16,), _SCBIN + 1, jnp.uint32)

        def process(buf):
            @plsc.parallel_loop(0, _CB // 16, unroll=_UNROLL)
            def _(i):
                v = buf[pl.ds(i * 16, 16)]
                bits = lax.bitcast_convert_type(v, jnp.int32)
                rel = lax.bitcast_convert_type(bits - lo, jnp.uint32)
                idu = jnp.minimum(rel >> shift, ovf)
                idx = lax.bitcast_convert_type(idu, jnp.int32) + laneoff
                plsc.addupdate_scatter(hist, [idx], ones)

        bufs = (bufa, bufb)
        sems = (sema, semb)
        pltpu.async_copy(s_hbm.at[pl.ds(base, _CB)], bufa, sema)

        for c in range(nchunks):
            cur = c % 2
            if c + 1 < nchunks:
                pltpu.async_copy(
                    s_hbm.at[pl.ds(base + (c + 1) * _CB, _CB)],
                    bufs[1 - cur], sems[1 - cur])
            pltpu.make_async_copy(
                s_hbm.at[pl.ds(base + c * _CB, _CB)], bufs[cur], sems[cur]
            ).wait()
            process(bufs[cur])

        @plsc.parallel_loop(0, _SCBIN // 16, unroll=2)
        def _(cgrp):
            acc = hist[pl.ds(cgrp * 16, 16)]
            for l in range(1, 16):
                acc = acc + hist[pl.ds(l * _LSTRIDE + cgrp * 16, 16)]
            merged[pl.ds(cgrp * 16, 16)] = acc

        pltpu.sync_copy(merged, out_hbm.at[pl.ds(wid * _SCBIN, _SCBIN)])

    return sc_hist


@jax.jit
def kernel(x):
    n, ch, hh, ww = x.shape
    imgs = n * ch
    xf = x.reshape(imgs, hh, ww)
    grid = (imgs // _G,)
    blk = pl.BlockSpec((_G, hh, ww), lambda i: (i, 0, 0))
    hspec = pl.BlockSpec((1, _NBIN), lambda i: (0, 0))
    sspec = pl.BlockSpec(memory_space=pltpu.SMEM)

    total = imgs * hh * ww
    k = jnp.int32(int(_TOPK_RATIO * total))

    s, h1 = pl.pallas_call(
        _score_body,
        grid=grid,
        in_specs=[blk],
        out_specs=[blk, hspec],
        out_shape=[
            jax.ShapeDtypeStruct((imgs, hh, ww), jnp.float32),
            jax.ShapeDtypeStruct((1, _NBIN), jnp.int32),
        ],
    )(xf)

    lo, k_rem = _refine(h1, jnp.int32(0), k, 27)

    s_flat = s.reshape(total)
    for shift in (15, 3):
        h = _make_sc_hist(total, shift)(s_flat, jnp.full((16,), lo, jnp.int32))
        lo, k_rem = _refine(h.reshape(32, _SCBIN).sum(0), lo, k_rem, shift,
                            nbin=_SCBIN)

    out = pl.pallas_call(
        _mask_body,
        grid=grid,
        in_specs=[sspec, blk, blk],
        out_specs=blk,
        out_shape=jax.ShapeDtypeStruct((imgs, hh, ww), jnp.float32),
    )(lo.reshape(1, 1), xf, s)

    return out.reshape(n, ch, hh, ww)


# speedup vs baseline: 74.7353x; 1.3893x over previous
import functools

import jax
import jax.numpy as jnp
from jax import lax
from jax.experimental import pallas as pl
from jax.experimental.pallas import tpu as pltpu
from jax.experimental.pallas import tpu_sc as plsc

_KERNEL = 7
_PAD = 3
_TOPK_RATIO = 0.1

_G = 16


def _shifted(v, d, axis, fill):
    n = v.shape[axis]
    if d == 0:
        return v
    pad = jnp.full_like(v, fill)
    if d > 0:
        a = jax.lax.slice_in_dim(v, d, n, axis=axis)
        b = jax.lax.slice_in_dim(pad, 0, d, axis=axis)
        return jax.lax.concatenate([a, b], axis)
    else:
        a = jax.lax.slice_in_dim(pad, 0, -d, axis=axis)
        b = jax.lax.slice_in_dim(v, 0, n + d, axis=axis)
        return jax.lax.concatenate([a, b], axis)


def _slide7(v, axis, op, fill):
    m3 = op(op(v, _shifted(v, 1, axis, fill)), _shifted(v, -1, axis, fill))
    return op(op(m3, _shifted(m3, 2, axis, fill)),
              _shifted(m3, -2, axis, fill))


def _pool_scores(v):
    rmax = _slide7(v, 2, jnp.maximum, -jnp.inf)
    rsum = _slide7(v, 2, jnp.add, 0.0)
    cmax = _slide7(rmax, 1, jnp.maximum, -jnp.inf)
    csum = _slide7(rsum, 1, jnp.add, 0.0)
    return jnp.abs(cmax - csum * (1.0 / float(_KERNEL * _KERNEL)))


def _score_body(x_ref, s_ref):
    s_ref[...] = _pool_scores(x_ref[...])


def _mask_body(t_ref, x_ref, s_ref, o_ref):
    bits = jax.lax.bitcast_convert_type(s_ref[...], jnp.int32)
    o_ref[...] = jnp.where(bits >= t_ref[0, 0], x_ref[...], 0.0)


def _refine(hist, lo, k, shift, nbin):
    h = hist.reshape(nbin)
    c = jnp.cumsum(h[::-1])[::-1]
    bstar = jnp.sum((c >= k).astype(jnp.int32)) - 1
    kept_above = c[bstar] - h[bstar]
    return lo + (bstar << shift), k - kept_above


_SCBIN = 4096
_LSTRIDE = _SCBIN + 2
_CB = 18816
_UNROLL = 8


def _make_sc_hist(total, shift):
    nw = 32
    chunk = total // nw
    nchunks = chunk // _CB
    assert chunk % _CB == 0 and chunk % 16 == 0
    mesh = plsc.VectorSubcoreMesh(core_axis_name="c", subcore_axis_name="s")

    @functools.partial(
        pl.kernel,
        mesh=mesh,
        compiler_params=pltpu.CompilerParams(needs_layout_passes=False),
        out_type=jax.ShapeDtypeStruct((nw * _SCBIN,), jnp.int32),
        scratch_types=[
            pltpu.VMEM((16 * _LSTRIDE,), jnp.int32),
            pltpu.VMEM((_CB,), jnp.float32),
            pltpu.VMEM((_CB,), jnp.float32),
            pltpu.VMEM((_SCBIN,), jnp.int32),
            pltpu.VMEM((16,), jnp.int32),
            pltpu.SemaphoreType.DMA,
            pltpu.SemaphoreType.DMA,
        ],
    )
    def sc_hist(s_hbm, lo_hbm, out_hbm, hist, bufa, bufb, merged, lov, sema, semb):
        wid = lax.axis_index("s") * 2 + lax.axis_index("c")
        base = wid * chunk

        zero16 = jnp.zeros((16,), jnp.int32)

        @plsc.parallel_loop(0, (16 * _LSTRIDE) // 16, unroll=8)
        def _(i):
            hist[pl.ds(i * 16, 16)] = zero16

        pltpu.sync_copy(lo_hbm, lov)
        lo = lov[...]
        laneoff = lax.iota(jnp.int32, 16) * _LSTRIDE
        ones = jnp.ones((16,), jnp.int32)
        ovf = jnp.full((16,), _SCBIN + 1, jnp.uint32)

        def process(buf):
            @plsc.parallel_loop(0, _CB // 16, unroll=_UNROLL)
            def _(i):
                v = buf[pl.ds(i * 16, 16)]
                bits = lax.bitcast_convert_type(v, jnp.int32)
                rel = lax.bitcast_convert_type(bits - lo, jnp.uint32)
                idu = jnp.minimum(rel >> shift, ovf)
                idx = lax.bitcast_convert_type(idu, jnp.int32) + laneoff
                plsc.addupdate_scatter(hist, [idx], ones)

        bufs = (bufa, bufb)
        sems = (sema, semb)
        pltpu.async_copy(s_hbm.at[pl.ds(base, _CB)], bufa, sema)

        for c in range(nchunks):
            cur = c % 2
            if c + 1 < nchunks:
                pltpu.async_copy(
                    s_hbm.at[pl.ds(base + (c + 1) * _CB, _CB)],
                    bufs[1 - cur], sems[1 - cur])
            pltpu.make_async_copy(
                s_hbm.at[pl.ds(base + c * _CB, _CB)], bufs[cur], sems[cur]
            ).wait()
            process(bufs[cur])

        @plsc.parallel_loop(0, _SCBIN // 16, unroll=2)
        def _(cgrp):
            acc = hist[pl.ds(cgrp * 16, 16)]
            for l in range(1, 16):
                acc = acc + hist[pl.ds(l * _LSTRIDE + cgrp * 16, 16)]
            merged[pl.ds(cgrp * 16, 16)] = acc

        pltpu.sync_copy(merged, out_hbm.at[pl.ds(wid * _SCBIN, _SCBIN)])

    return sc_hist


@jax.jit
def kernel(x):
    n, ch, hh, ww = x.shape
    imgs = n * ch
    xf = x.reshape(imgs, hh, ww)
    grid = (imgs // _G,)
    blk = pl.BlockSpec((_G, hh, ww), lambda i: (i, 0, 0))
    sspec = pl.BlockSpec(memory_space=pltpu.SMEM)

    total = imgs * hh * ww
    k = jnp.int32(int(_TOPK_RATIO * total))

    s = pl.pallas_call(
        _score_body,
        grid=grid,
        in_specs=[blk],
        out_specs=blk,
        out_shape=jax.ShapeDtypeStruct((imgs, hh, ww), jnp.float32),
    )(xf)

    lo, k_rem = jnp.int32(0), k

    s_flat = s.reshape(total)
    for shift in (19, 7, 0):
        h = _make_sc_hist(total, shift)(s_flat, jnp.full((16,), lo, jnp.int32))
        lo, k_rem = _refine(h.reshape(32, _SCBIN).sum(0), lo, k_rem, shift,
                            nbin=_SCBIN)

    out = pl.pallas_call(
        _mask_body,
        grid=grid,
        in_specs=[sspec, blk, blk],
        out_specs=blk,
        out_shape=jax.ShapeDtypeStruct((imgs, hh, ww), jnp.float32),
    )(lo.reshape(1, 1), xf, s)

    return out.reshape(n, ch, hh, ww)
